# baseline (speedup 1.0000x reference)
import functools

import jax
import jax.numpy as jnp
from jax import lax
from jax.experimental import pallas as pl
from jax.experimental.pallas import tpu as pltpu
from jax.experimental.pallas import tpu_sc as plsc

N_EMBD = 64
CODEBOOK = 100000
G = 8

NC, NS = 2, 16
NW = NC * NS
CHUNK = 128
NBUF = 2
L = 16


def kernel(idx, tables):
    B, T, g_dim = idx.shape
    N = B * T
    per_w = N // NW
    n_chunks = per_w // CHUNK

    idx2 = idx.reshape(N, G).astype(jnp.int32)
    table_flat = tables.reshape(G * CODEBOOK, N_EMBD)

    mesh = plsc.VectorSubcoreMesh(core_axis_name="c", subcore_axis_name="s")

    @functools.partial(
        pl.kernel,
        out_type=jax.ShapeDtypeStruct((N, N_EMBD), jnp.float32),
        mesh=mesh,
        compiler_params=pltpu.CompilerParams(use_tc_tiling_on_sc=False,
                                             needs_layout_passes=False),
        scratch_types=[
            pltpu.VMEM((NBUF, CHUNK, G), jnp.int32),
            pltpu.VMEM((NBUF, G, CHUNK), jnp.int32),
            pltpu.VMEM((NBUF, CHUNK, N_EMBD), jnp.float32),
            pltpu.SemaphoreType.DMA((NBUF,)),
            pltpu.SemaphoreType.DMA((NBUF,)),
        ],
    )
    def body(idx_hbm, tab_hbm, out_hbm, raw_v, idx_v, acc_v, sem_idx,
             sem_acc):
        wid = lax.axis_index("s") * NC + lax.axis_index("c")
        tok0 = wid * per_w
        lane = lax.iota(jnp.int32, L)

        def zero_acc(b):
            @pl.loop(0, CHUNK)
            def _(r):
                for k in range(N_EMBD // L):
                    acc_v[b, r, pl.ds(k * L, L)] = jnp.zeros(
                        (L,), jnp.float32)

        def transpose_idx(b):
            for g in range(G):
                col = jnp.full((L,), g, jnp.int32)
                off = jnp.full((L,), g * CODEBOOK, jnp.int32)
                for k in range(CHUNK // L):
                    rows = lane + (k * L)
                    vals = plsc.load_gather(raw_v.at[b], [rows, col])
                    idx_v[b, g, pl.ds(k * L, L)] = vals + off

        def fire_gathers(b):
            for g in range(G):
                pltpu.async_copy(tab_hbm.at[idx_v.at[b, g]], acc_v.at[b],
                                 sem_acc.at[b], add=True)

        def drain_gathers(b):
            for g in range(G):
                pltpu.make_async_copy(tab_hbm.at[idx_v.at[b, g]],
                                      acc_v.at[b], sem_acc.at[b]).wait()

        def copy_out(b, j):
            pltpu.sync_copy(acc_v.at[b],
                            out_hbm.at[pl.ds(tok0 + j * CHUNK, CHUNK)])

        for b in range(NBUF):
            zero_acc(b)
            pltpu.sync_copy(idx_hbm.at[pl.ds(tok0 + b * CHUNK, CHUNK)],
                            raw_v.at[b])
            transpose_idx(b)
            fire_gathers(b)

        @pl.loop(0, n_chunks - NBUF, step=NBUF)
        def _(jj):
            for b in range(NBUF):
                j = jj + b
                drain_gathers(b)
                idx_cp = pltpu.async_copy(
                    idx_hbm.at[pl.ds(tok0 + (j + NBUF) * CHUNK, CHUNK)],
                    raw_v.at[b], sem_idx.at[b])
                copy_out(b, j)
                zero_acc(b)
                idx_cp.wait()
                transpose_idx(b)
                fire_gathers(b)

        for b in range(NBUF):
            drain_gathers(b)
            copy_out(b, n_chunks - NBUF + b)

    out = body(idx2, table_flat)
    return out.reshape(B, T, N_EMBD)


# speedup vs baseline: 1.0608x; 1.0608x over previous
import functools

import jax
import jax.numpy as jnp
from jax import lax
from jax.experimental import pallas as pl
from jax.experimental.pallas import tpu as pltpu
from jax.experimental.pallas import tpu_sc as plsc

N_EMBD = 64
CODEBOOK = 100000
G = 8

NC, NS = 2, 16
NW = NC * NS
BCH = 8
NBUF = 2
L = 16


def kernel(idx, tables):
    B, T, g_dim = idx.shape
    N = B * T
    rows_w = B // NW
    n_chunks = rows_w // BCH
    CHUNK = BCH * T

    mesh = plsc.VectorSubcoreMesh(core_axis_name="c", subcore_axis_name="s")

    @functools.partial(
        pl.kernel,
        out_type=jax.ShapeDtypeStruct((N, N_EMBD), jnp.float32),
        mesh=mesh,
        compiler_params=pltpu.CompilerParams(use_tc_tiling_on_sc=False,
                                             needs_layout_passes=False),
        scratch_types=[
            pltpu.VMEM((NBUF, BCH, T, G), jnp.int32),
            pltpu.VMEM((NBUF, G, CHUNK), jnp.int32),
            pltpu.VMEM((NBUF, CHUNK, N_EMBD), jnp.float32),
            pltpu.SemaphoreType.DMA((NBUF,)),
            pltpu.SemaphoreType.DMA((NBUF,)),
        ],
    )
    def body(idx_hbm, tab_hbm, out_hbm, raw_v, idx_v, acc_v, sem_idx,
             sem_acc):
        wid = lax.axis_index("s") * NC + lax.axis_index("c")
        row0 = wid * rows_w
        lane = lax.iota(jnp.int32, L)

        def zero_acc(b):
            @pl.loop(0, CHUNK)
            def _(r):
                for k in range(N_EMBD // L):
                    acc_v[b, r, pl.ds(k * L, L)] = jnp.zeros(
                        (L,), jnp.float32)

        def transpose_idx(b):
            for k in range(CHUNK // L):
                t = lane + (k * L)
                b_i = t // T
                t_i = t - b_i * T
                for g in range(G):
                    col = jnp.full((L,), g, jnp.int32)
                    vals = plsc.load_gather(raw_v.at[b], [b_i, t_i, col])
                    idx_v[b, g, pl.ds(k * L, L)] = vals

        def fire_gathers(b):
            for g in range(G):
                pltpu.async_copy(tab_hbm.at[g].at[idx_v.at[b, g]],
                                 acc_v.at[b], sem_acc.at[b], add=True)

        def drain_gathers(b):
            for g in range(G):
                pltpu.make_async_copy(tab_hbm.at[g].at[idx_v.at[b, g]],
                                      acc_v.at[b], sem_acc.at[b]).wait()

        def copy_out(b, j):
            pltpu.sync_copy(
                acc_v.at[b],
                out_hbm.at[pl.ds((row0 + j * BCH) * T, CHUNK)])

        for b in range(NBUF):
            zero_acc(b)
            pltpu.sync_copy(idx_hbm.at[pl.ds(row0 + b * BCH, BCH)],
                            raw_v.at[b])
            transpose_idx(b)
            fire_gathers(b)

        @pl.loop(0, n_chunks - NBUF, step=NBUF)
        def _(jj):
            for b in range(NBUF):
                j = jj + b
                drain_gathers(b)
                idx_cp = pltpu.async_copy(
                    idx_hbm.at[pl.ds(row0 + (j + NBUF) * BCH, BCH)],
                    raw_v.at[b], sem_idx.at[b])
                copy_out(b, j)
                zero_acc(b)
                idx_cp.wait()
                transpose_idx(b)
                fire_gathers(b)

        for b in range(NBUF):
            drain_gathers(b)
            copy_out(b, n_chunks - NBUF + b)

    out = body(idx, tables)
    return out.reshape(B, T, N_EMBD)


# speedup vs baseline: 1.0636x; 1.0027x over previous
import functools

import jax
import jax.numpy as jnp
from jax import lax
from jax.experimental import pallas as pl
from jax.experimental.pallas import tpu as pltpu
from jax.experimental.pallas import tpu_sc as plsc

N_EMBD = 64
CODEBOOK = 100000
G = 8

NC, NS = 2, 16
NW = NC * NS
LB = 128
NBUF = 2
L = 16


def kernel(idx, tables):
    B, T, g_dim = idx.shape
    BT = B // LB
    n_blocks = BT * T
    per_w = n_blocks // NW
    EH, EL = N_EMBD // 8, 8

    idx_w = (idx.transpose(1, 2, 0)
                .reshape(T, g_dim, BT, LB)
                .transpose(0, 2, 1, 3))

    mesh = plsc.VectorSubcoreMesh(core_axis_name="c", subcore_axis_name="s")

    @functools.partial(
        pl.kernel,
        out_type=jax.ShapeDtypeStruct((T, EH, BT, EL, LB), jnp.float32),
        mesh=mesh,
        compiler_params=pltpu.CompilerParams(use_tc_tiling_on_sc=False,
                                             needs_layout_passes=False),
        scratch_types=[
            pltpu.VMEM((NBUF, G, LB), jnp.int32),
            pltpu.VMEM((NBUF, LB, N_EMBD), jnp.float32),
            pltpu.VMEM((NBUF, EH, EL, LB), jnp.float32),
            pltpu.SemaphoreType.DMA((NBUF,)),
            pltpu.SemaphoreType.DMA((NBUF,)),
            pltpu.SemaphoreType.DMA((NBUF,)),
        ],
    )
    def body(idx_hbm, tab_hbm, out_hbm, idx_v, acc_v, acct_v, sem_idx,
             sem_acc, sem_out):
        wid = lax.axis_index("s") * NC + lax.axis_index("c")
        blk0 = wid * per_w
        lane = lax.iota(jnp.int32, L)

        def zero_acc(b):
            @pl.loop(0, LB)
            def _(r):
                for k in range(N_EMBD // L):
                    acc_v[b, r, pl.ds(k * L, L)] = jnp.zeros(
                        (L,), jnp.float32)

        def fire_gathers(b):
            for g in range(G):
                pltpu.async_copy(tab_hbm.at[g].at[idx_v.at[b, g]],
                                 acc_v.at[b], sem_acc.at[b], add=True)

        def drain_gathers(b):
            for g in range(G):
                pltpu.make_async_copy(tab_hbm.at[g].at[idx_v.at[b, g]],
                                      acc_v.at[b], sem_acc.at[b]).wait()

        def transpose_acc(b):
            @pl.loop(0, EH)
            def _(eh):
                for el in range(EL):
                    col = jnp.full((L,), 0, jnp.int32) + (eh * EL + el)
                    for k in range(LB // L):
                        rows = lane + (k * L)
                        vals = plsc.load_gather(acc_v.at[b], [rows, col])
                        acct_v[b, eh, el, pl.ds(k * L, L)] = vals

        def copy_out(b, blk):
            t = blk // BT
            bt = blk - t * BT
            return pltpu.async_copy(acct_v.at[b],
                                    out_hbm.at[t, :, bt], sem_out.at[b])

        for b in range(NBUF):
            zero_acc(b)
            blk = blk0 + b
            t = blk // BT
            bt = blk - t * BT
            pltpu.sync_copy(idx_hbm.at[t, bt], idx_v.at[b])
            fire_gathers(b)

        @pl.loop(0, per_w - NBUF, step=NBUF)
        def _(jj):
            for b in range(NBUF):
                blk = blk0 + jj + b
                nblk = blk + NBUF
                nt = nblk // BT
                nbt = nblk - nt * BT
                drain_gathers(b)
                idx_cp = pltpu.async_copy(idx_hbm.at[nt, nbt], idx_v.at[b],
                                          sem_idx.at[b])
                @pl.when(jj > 0)
                def _wait_prev_out():
                    pltpu.make_async_copy(acct_v.at[b], out_hbm.at[0, :, 0],
                                          sem_out.at[b]).wait()
                transpose_acc(b)
                copy_out(b, blk)
                zero_acc(b)
                idx_cp.wait()
                fire_gathers(b)

        for b in range(NBUF):
            pltpu.make_async_copy(acct_v.at[b], out_hbm.at[0, :, 0],
                                  sem_out.at[b]).wait()
            drain_gathers(b)
            transpose_acc(b)
            copy_out(b, per_w - NBUF + b + blk0).wait()

    out5 = body(idx_w, tables)
    return (out5.transpose(2, 4, 0, 1, 3)
                .reshape(B, T, N_EMBD))
